# baseline (speedup 1.0000x reference)
import functools
import math

import jax
import jax.numpy as jnp
from jax.experimental import pallas as pl
from jax.experimental.pallas import tpu as pltpu
from jax.experimental.pallas import tpu_sc as plsc

_N = 5000
_NP = 5120
_B = 1024
_NB = _NP // _B
_TOP = 100
_TOPP = 104
_SCORE_THR = 0.05
_IOU_THR = 0.5
_CW = 1333.0
_CH = 800.0
_CLIP = float(math.log(1000.0 / 16.0))


def _prep_kernel(r0, r1, r2, r3, p0, p1, p2, p3, s, cf,
                 x1o, y1o, x2o, y2o, ox1o, oy1o, ox2o, oy2o, area_o, valid_o):
    dx = r0[...] * 0.1
    dy = r1[...] * 0.1
    dw = jnp.minimum(r2[...] * 0.2, _CLIP)
    dh = jnp.minimum(r3[...] * 0.2, _CLIP)
    cx = p0[...] + dx * p2[...]
    cy = p1[...] + dy * p3[...]
    w = p2[...] * jnp.exp(dw)
    h = p3[...] * jnp.exp(dh)
    x1 = jnp.clip(cx - 0.5 * w, 0.0, _CW)
    y1 = jnp.clip(cy - 0.5 * h, 0.0, _CH)
    x2 = jnp.clip(cx + 0.5 * w, 0.0, _CW)
    y2 = jnp.clip(cy + 0.5 * h, 0.0, _CH)
    valid = ((x2 - x1) > 0.0) & ((y2 - y1) > 0.0) & (s[...] > _SCORE_THR)
    off = cf[...] * (_CW + 1.0)
    ox1 = jnp.where(valid, x1 + off, 2e9)
    oy1 = jnp.where(valid, y1 + off, 2e9)
    ox2 = jnp.where(valid, x2 + off, 2e9)
    oy2 = jnp.where(valid, y2 + off, 2e9)
    x1o[...] = x1
    y1o[...] = y1
    x2o[...] = x2
    y2o[...] = y2
    ox1o[...] = ox1
    oy1o[...] = oy1
    ox2o[...] = ox2
    oy2o[...] = oy2
    area_o[...] = (ox2 - ox1) * (oy2 - oy1)
    valid_o[...] = valid.astype(jnp.float32)


def _sup_kernel(ox1r, oy1r, ox2r, oy2r, ar, sr, ir,
                ox1c, oy1c, ox2c, oy2c, ac, sc_, ic,
                out_r, out_c, scr_r, scr_c):
    r = pl.program_id(0)
    c = pl.program_id(1)

    @pl.when((r == 0) & (c == 0))
    def _zero():
        scr_r[...] = jnp.zeros_like(scr_r)
        scr_c[...] = jnp.zeros_like(scr_c)

    @pl.when(c >= r)
    def _tile():
        ltx = jnp.maximum(ox1r[...], ox1c[...])
        lty = jnp.maximum(oy1r[...], oy1c[...])
        rbx = jnp.minimum(ox2r[...], ox2c[...])
        rby = jnp.minimum(oy2r[...], oy2c[...])
        ww = jnp.maximum(rbx - ltx, 0.0)
        hh = jnp.maximum(rby - lty, 0.0)
        inter = ww * hh
        union = ar[...] + ac[...] - inter
        iou = inter / (union + 1e-9)
        gt = iou > _IOU_THR
        eq_idx = ic[...] == ir[...]
        hcr = (sc_[...] > sr[...]) | ((sc_[...] == sr[...]) & (ic[...] < ir[...]))
        hrc = ~(hcr | eq_idx)
        acc_r = jnp.any(gt & hcr, axis=1, keepdims=True).astype(jnp.float32)
        acc_c = jnp.any(gt & hrc, axis=0, keepdims=True).astype(jnp.float32)
        scr_r[pl.ds(r, 1)] = jnp.maximum(scr_r[pl.ds(r, 1)],
                                         acc_r.reshape(1, _B, 1))
        scr_c[pl.ds(c, 1)] = jnp.maximum(scr_c[pl.ds(c, 1)],
                                         acc_c.reshape(1, 1, _B))

    @pl.when((r == _NB - 1) & (c == _NB - 1))
    def _emit():
        out_r[...] = scr_r[...]
        out_c[...] = scr_c[...]


def _sel_kernel(sc_, vc, sup_a, sup_b, ic, out_idx):
    valid = vc[...] > 0.5
    sup = (sup_a[...] + sup_b[...]) > 0.5
    s = sc_[...]
    idx = ic[...]
    real = idx < float(_N)
    c = jnp.where(valid & ~sup, s, jnp.where(valid, s - 2.0, -3.0))
    c = jnp.where(real, c, -4.0)

    def body(k, cval):
        m = jnp.max(cval)
        isel = jnp.min(jnp.where(cval == m, idx, float(_NP)))
        out_idx[pl.ds(k, 1), :] = isel.reshape(1, 1)
        return jnp.where(idx == isel, -1e9, cval)

    jax.lax.fori_loop(0, _TOP, body, c)
    out_idx[pl.ds(_TOP, _TOPP - _TOP), :] = jnp.zeros((_TOPP - _TOP, 1),
                                                      jnp.float32)


_SC_INFO = plsc.get_sparse_core_info()
_NW = _SC_INFO.num_cores * _SC_INFO.num_subcores
_GB = 256
_GD = 128
_BPW = _GB // _NW


def _make_sc_gather():
    mesh = plsc.VectorSubcoreMesh(core_axis_name="c", subcore_axis_name="s")

    @functools.partial(
        pl.kernel, mesh=mesh,
        out_type=jax.ShapeDtypeStruct((_GB, _GD), jnp.float32),
        scratch_types=[
            pltpu.VMEM((_BPW,), jnp.int32),
            pltpu.VMEM((_BPW, _GD), jnp.float32),
            pltpu.SemaphoreType.DMA,
        ],
    )
    def k(table_hbm, idx_hbm, out_hbm, idx_v, rows_v, sem):
        wid = jax.lax.axis_index("s") * _SC_INFO.num_cores + jax.lax.axis_index("c")
        base = wid * _BPW
        pltpu.sync_copy(idx_hbm.at[pl.ds(base, _BPW)], idx_v)
        pltpu.async_copy(table_hbm.at[idx_v], rows_v, sem).wait()
        pltpu.sync_copy(rows_v, out_hbm.at[pl.ds(base, _BPW)])

    return k


def kernel(reg, proposals, scores, classes):
    pad = _NP - _N
    regp = jnp.pad(reg, ((0, pad), (0, 0)))
    prp = jnp.pad(proposals, ((0, pad), (0, 0)))
    sp = jnp.pad(scores, (0, pad)).reshape(1, _NP)
    cfp = jnp.pad(classes.astype(jnp.float32), (0, pad)).reshape(1, _NP)
    iota = jnp.arange(_NP, dtype=jnp.float32).reshape(1, _NP)
    r0, r1, r2, r3 = (regp[:, i].reshape(1, _NP) for i in range(4))
    p0, p1, p2, p3 = (prp[:, i].reshape(1, _NP) for i in range(4))

    vec = jax.ShapeDtypeStruct((1, _NP), jnp.float32)
    x1, y1, x2, y2, ox1, oy1, ox2, oy2, area, validf = pl.pallas_call(
        _prep_kernel,
        out_shape=[vec] * 10,
    )(r0, r1, r2, r3, p0, p1, p2, p3, sp, cfp)

    col = lambda a: a.reshape(_NP, 1)
    row_spec = pl.BlockSpec((_B, 1), lambda r, c: (r, 0))
    col_spec = pl.BlockSpec((1, _B), lambda r, c: (0, jnp.maximum(c, r)))
    full_r = pl.BlockSpec((_NB, _B, 1), lambda r, c: (0, 0, 0))
    full_c = pl.BlockSpec((_NB, 1, _B), lambda r, c: (0, 0, 0))
    sup_r, sup_c = pl.pallas_call(
        _sup_kernel,
        grid=(_NB, _NB),
        in_specs=[row_spec] * 7 + [col_spec] * 7,
        out_specs=[full_r, full_c],
        out_shape=[jax.ShapeDtypeStruct((_NB, _B, 1), jnp.float32),
                   jax.ShapeDtypeStruct((_NB, 1, _B), jnp.float32)],
        scratch_shapes=[pltpu.VMEM((_NB, _B, 1), jnp.float32),
                        pltpu.VMEM((_NB, 1, _B), jnp.float32)],
    )(col(ox1), col(oy1), col(ox2), col(oy2), col(area), col(sp), col(iota),
      ox1, oy1, ox2, oy2, area, sp, iota)

    sel_idx = pl.pallas_call(
        _sel_kernel,
        out_shape=jax.ShapeDtypeStruct((_TOPP, 1), jnp.float32),
    )(sp, validf, sup_r.reshape(1, _NP), sup_c.reshape(1, _NP), iota)

    table = jnp.concatenate(
        [x1.reshape(_NP, 1), y1.reshape(_NP, 1), x2.reshape(_NP, 1),
         y2.reshape(_NP, 1), sp.reshape(_NP, 1), cfp.reshape(_NP, 1),
         jnp.zeros((_NP, _GD - 6), jnp.float32)], axis=1)
    idxp = jnp.pad(sel_idx[:_TOP, 0].astype(jnp.int32), (0, _GB - _TOP))
    g = _make_sc_gather()(table, idxp)

    sel_boxes = g[:_TOP, 0:4]
    sel_scores = g[:_TOP, 4]
    sel_classes = g[:_TOP, 5].astype(jnp.int32)
    return sel_boxes, sel_scores, sel_classes


# speedup vs baseline: 1.3383x; 1.3383x over previous
import math

import jax
import jax.numpy as jnp
from jax.experimental import pallas as pl
from jax.experimental.pallas import tpu as pltpu

_N = 5000
_NP = 5120
_B = 1024
_NB = _NP // _B
_TOP = 100
_TOPP = 104
_SCORE_THR = 0.05
_IOU_THR = 0.5
_CW = 1333.0
_CH = 800.0
_CLIP = float(math.log(1000.0 / 16.0))


def _prep_kernel(r0, r1, r2, r3, p0, p1, p2, p3, s, cf,
                 x1o, y1o, x2o, y2o, ox1o, oy1o, ox2o, oy2o, area_o, valid_o):
    dx = r0[...] * 0.1
    dy = r1[...] * 0.1
    dw = jnp.minimum(r2[...] * 0.2, _CLIP)
    dh = jnp.minimum(r3[...] * 0.2, _CLIP)
    cx = p0[...] + dx * p2[...]
    cy = p1[...] + dy * p3[...]
    w = p2[...] * jnp.exp(dw)
    h = p3[...] * jnp.exp(dh)
    x1 = jnp.clip(cx - 0.5 * w, 0.0, _CW)
    y1 = jnp.clip(cy - 0.5 * h, 0.0, _CH)
    x2 = jnp.clip(cx + 0.5 * w, 0.0, _CW)
    y2 = jnp.clip(cy + 0.5 * h, 0.0, _CH)
    valid = ((x2 - x1) > 0.0) & ((y2 - y1) > 0.0) & (s[...] > _SCORE_THR)
    off = cf[...] * (_CW + 1.0)
    ox1 = jnp.where(valid, x1 + off, 2e9)
    oy1 = jnp.where(valid, y1 + off, 2e9)
    ox2 = jnp.where(valid, x2 + off, 2e9)
    oy2 = jnp.where(valid, y2 + off, 2e9)
    x1o[...] = x1
    y1o[...] = y1
    x2o[...] = x2
    y2o[...] = y2
    ox1o[...] = ox1
    oy1o[...] = oy1
    ox2o[...] = ox2
    oy2o[...] = oy2
    area_o[...] = (ox2 - ox1) * (oy2 - oy1)
    valid_o[...] = valid.astype(jnp.float32)


def _sup_kernel(ox1r, oy1r, ox2r, oy2r, ar, sr, ir,
                ox1c, oy1c, ox2c, oy2c, ac, sc_, ic,
                sf, vf, i_f, vals, out, scr_r, scr_c, oh_ref):
    r = pl.program_id(0)
    c = pl.program_id(1)

    @pl.when((r == 0) & (c == 0))
    def _zero():
        scr_r[...] = jnp.zeros_like(scr_r)
        scr_c[...] = jnp.zeros_like(scr_c)

    @pl.when(c >= r)
    def _tile():
        ltx = jnp.maximum(ox1r[...], ox1c[...])
        lty = jnp.maximum(oy1r[...], oy1c[...])
        rbx = jnp.minimum(ox2r[...], ox2c[...])
        rby = jnp.minimum(oy2r[...], oy2c[...])
        ww = jnp.maximum(rbx - ltx, 0.0)
        hh = jnp.maximum(rby - lty, 0.0)
        inter = ww * hh
        union = ar[...] + ac[...] - inter
        iou = inter / (union + 1e-9)
        gt = iou > _IOU_THR
        eq_idx = ic[...] == ir[...]
        hcr = (sc_[...] > sr[...]) | ((sc_[...] == sr[...]) & (ic[...] < ir[...]))
        hrc = ~(hcr | eq_idx)
        acc_r = jnp.any(gt & hcr, axis=1, keepdims=True).astype(jnp.float32)
        acc_c = jnp.any(gt & hrc, axis=0, keepdims=True).astype(jnp.float32)
        acc_rt = jnp.transpose(acc_r, (1, 0))
        scr_r[pl.ds(r, 1)] = jnp.maximum(scr_r[pl.ds(r, 1)],
                                         acc_rt.reshape(1, 1, _B))
        scr_c[pl.ds(c, 1)] = jnp.maximum(scr_c[pl.ds(c, 1)],
                                         acc_c.reshape(1, 1, _B))

    @pl.when((r == _NB - 1) & (c == _NB - 1))
    def _select():
        sup_a = jnp.concatenate(
            [scr_r[i].reshape(1, _B) for i in range(_NB)], axis=1)
        sup_b = jnp.concatenate(
            [scr_c[i].reshape(1, _B) for i in range(_NB)], axis=1)
        valid = vf[...] > 0.5
        sup = (sup_a + sup_b) > 0.5
        s = sf[...]
        idx = i_f[...]
        real = idx < float(_N)
        key = jnp.where(valid & ~sup, s, jnp.where(valid, s - 2.0, -3.0))
        key = jnp.where(real, key, -4.0)

        oh_ref[...] = jnp.zeros_like(oh_ref)

        def body(k, cval):
            m = jnp.max(cval)
            isel = jnp.min(jnp.where(cval == m, idx, float(_NP)))
            onehot = idx == isel
            oh_ref[pl.ds(k, 1), :] = onehot.astype(jnp.float32)
            return jnp.where(onehot, -1e9, cval)

        jax.lax.fori_loop(0, _TOP, body, key)

        out[...] = jnp.dot(oh_ref[...], vals[...],
                           preferred_element_type=jnp.float32,
                           precision=jax.lax.Precision.HIGHEST)


def kernel(reg, proposals, scores, classes):
    pad = _NP - _N
    regp = jnp.pad(reg, ((0, pad), (0, 0)))
    prp = jnp.pad(proposals, ((0, pad), (0, 0)))
    sp = jnp.pad(scores, (0, pad)).reshape(1, _NP)
    cfp = jnp.pad(classes.astype(jnp.float32), (0, pad)).reshape(1, _NP)
    iota = jnp.arange(_NP, dtype=jnp.float32).reshape(1, _NP)
    r0, r1, r2, r3 = (regp[:, i].reshape(1, _NP) for i in range(4))
    p0, p1, p2, p3 = (prp[:, i].reshape(1, _NP) for i in range(4))

    vec = jax.ShapeDtypeStruct((1, _NP), jnp.float32)
    x1, y1, x2, y2, ox1, oy1, ox2, oy2, area, validf = pl.pallas_call(
        _prep_kernel,
        out_shape=[vec] * 10,
    )(r0, r1, r2, r3, p0, p1, p2, p3, sp, cfp)

    vals = jnp.concatenate(
        [x1.reshape(_NP, 1), y1.reshape(_NP, 1), x2.reshape(_NP, 1),
         y2.reshape(_NP, 1), sp.reshape(_NP, 1), cfp.reshape(_NP, 1),
         jnp.zeros((_NP, 2), jnp.float32)], axis=1)

    col = lambda a: a.reshape(_NP, 1)
    row_spec = pl.BlockSpec((_B, 1), lambda r, c: (r, 0))
    col_spec = pl.BlockSpec((1, _B), lambda r, c: (0, jnp.maximum(c, r)))
    full_vec = pl.BlockSpec((1, _NP), lambda r, c: (0, 0))
    full_vals = pl.BlockSpec((_NP, 8), lambda r, c: (0, 0))
    sel = pl.pallas_call(
        _sup_kernel,
        grid=(_NB, _NB),
        in_specs=[row_spec] * 7 + [col_spec] * 7 + [full_vec] * 3 + [full_vals],
        out_specs=pl.BlockSpec((_TOPP, 8), lambda r, c: (0, 0)),
        out_shape=jax.ShapeDtypeStruct((_TOPP, 8), jnp.float32),
        scratch_shapes=[pltpu.VMEM((_NB, 1, _B), jnp.float32),
                        pltpu.VMEM((_NB, 1, _B), jnp.float32),
                        pltpu.VMEM((_TOPP, _NP), jnp.float32)],
    )(col(ox1), col(oy1), col(ox2), col(oy2), col(area), col(sp), col(iota),
      ox1, oy1, ox2, oy2, area, sp, iota,
      sp, validf, iota, vals)

    sel_boxes = sel[:_TOP, 0:4]
    sel_scores = sel[:_TOP, 4]
    sel_classes = sel[:_TOP, 5].astype(jnp.int32)
    return sel_boxes, sel_scores, sel_classes
